# baseline (speedup 1.0000x reference)
import functools

import jax
import jax.numpy as jnp
from jax import lax
from jax.experimental import pallas as pl
from jax.experimental.pallas import tpu as pltpu
from jax.experimental.pallas import tpu_sc as plsc

N_NODES = 10000
N_EDGES = 320000
D = 128
N_GRAPHS = 64
N_CLASSES = 10
BN_EPS = 1e-5

NC = 2
NS = 16
HALF = 5120
N_PAD = 10240
DUMP = N_NODES
A_PAD = 6144
CHUNK = 128
CHUNKS_PER_T = 157
E_PAD = NS * CHUNKS_PER_T * CHUNK
AROWS_PER_S = A_PAD // NS
OROWS_PER_S = HALF // NS
HROWS_PER_S = HALF // NS


def _segsum_body(h_hbm, src_hbm, dst_hbm, out_hbm,
                 sraw0, sraw1, draw0, draw1, gsrc0, gsrc1, gdst0, gdst1,
                 rows0, rows1, hs, acc,
                 gsem0, gsem1, ssem0, ssem1, isem0, isem1):
    c = lax.axis_index("c")
    s = lax.axis_index("s")
    base = s * CHUNKS_PER_T
    last = CHUNKS_PER_T - 1
    hbase = c * HALF
    dump_vec = HALF + s * 16 + lax.iota(jnp.int32, 16)

    sraws = (sraw0, sraw1)
    draws = (draw0, draw1)
    gsrcs = (gsrc0, gsrc1)
    gdsts = (gdst0, gdst1)
    rows = (rows0, rows1)
    gsems = (gsem0, gsem1)
    ssems = (ssem0, ssem1)
    isems = (isem0, isem1)

    pltpu.sync_copy(h_hbm.at[pl.ds(hbase + s * HROWS_PER_S, HROWS_PER_S), :],
                    hs.at[pl.ds(s * HROWS_PER_S, HROWS_PER_S), :])

    zv = jnp.zeros((16,), jnp.float32)

    def _zrow(r, carry):
        for k in range(D // 16):
            rows0[r, pl.ds(k * 16, 16)] = zv
        return carry

    def _round(rnd):
        d = c if rnd == 0 else 1 - c
        dbase = d * HALF

        lax.fori_loop(0, CHUNK, _zrow, 0)

        def _zacc(j, carry):
            pltpu.sync_copy(
                rows0, acc.at[pl.ds(s * AROWS_PER_S + j * CHUNK, CHUNK), :])
            return carry

        lax.fori_loop(0, AROWS_PER_S // CHUNK, _zacc, 0)
        plsc.subcore_barrier()

        def _mask(p, b):
            for k in range(D // 16):
                sl = pl.ds(k * 16, 16)
                loc = sraws[p][sl] - hbase
                dloc = draws[p][sl] - dbase
                valid = ((loc >= 0) & (loc < HALF)
                         & (dloc >= 0) & (dloc < HALF))
                gsrcs[b][sl] = jnp.where(valid, loc, 0)
                gdsts[b][sl] = jnp.where(valid, dloc, dump_vec)

        def _iter(j, b, first):
            nb = 1 - b
            if not first:
                pltpu.make_async_copy(h_hbm.at[pl.ds(0, CHUNK), :],
                                      rows[nb], ssems[nb]).wait()
            nj = jnp.minimum(j + 1, last)
            pltpu.make_async_copy(src_hbm.at[base + nj], sraws[nb],
                                  isems[nb]).wait()
            pltpu.make_async_copy(dst_hbm.at[base + nj], draws[nb],
                                  isems[nb]).wait()
            _mask(nb, nb)
            pltpu.async_copy(hs.at[gsrcs[nb]], rows[nb], gsems[nb])
            nj2 = jnp.minimum(j + 2, last)
            pltpu.async_copy(src_hbm.at[base + nj2], sraws[b], isems[b])
            pltpu.async_copy(dst_hbm.at[base + nj2], draws[b], isems[b])
            pltpu.make_async_copy(h_hbm.at[pl.ds(0, CHUNK), :],
                                  rows[b], gsems[b]).wait()
            pltpu.async_copy(rows[b], acc.at[gdsts[b]], ssems[b], add=True)

        pltpu.sync_copy(src_hbm.at[base], sraw0)
        pltpu.sync_copy(dst_hbm.at[base], draw0)
        _mask(0, 0)
        pltpu.async_copy(hs.at[gsrcs[0]], rows0, gsem0)
        pltpu.async_copy(src_hbm.at[base + 1], sraw1, isem1)
        pltpu.async_copy(dst_hbm.at[base + 1], draw1, isem1)

        _iter(0, 0, True)

        def _pair(j2, carry):
            for u in range(2):
                _iter(2 * j2 + 1 + u, (1 + u) % 2, False)
            return carry

        lax.fori_loop(0, (CHUNKS_PER_T - 1) // 2, _pair, 0)
        pltpu.make_async_copy(h_hbm.at[pl.ds(0, CHUNK), :], rows0, ssem0).wait()
        pltpu.make_async_copy(h_hbm.at[pl.ds(0, CHUNK), :], rows1, gsem1).wait()
        pltpu.make_async_copy(src_hbm.at[base], sraw0, isem0).wait()
        pltpu.make_async_copy(dst_hbm.at[base], draw0, isem0).wait()
        plsc.subcore_barrier()

        def _out(j, carry):
            r0 = s * OROWS_PER_S + j * 64
            pltpu.sync_copy(acc.at[pl.ds(r0, 64), :], rows1.at[pl.ds(0, 64), :])
            pltpu.sync_copy(rows1.at[pl.ds(0, 64), :],
                            out_hbm.at[c, pl.ds(dbase + r0, 64), :])
            return carry

        lax.fori_loop(0, OROWS_PER_S // 64, _out, 0)
        plsc.subcore_barrier()

    _round(0)
    _round(1)


@functools.lru_cache(maxsize=1)
def _get_segsum():
  return pl.kernel(
    _segsum_body,
    mesh=plsc.VectorSubcoreMesh(core_axis_name="c", subcore_axis_name="s"),
    out_type=jax.ShapeDtypeStruct((NC, N_PAD, D), jnp.float32),
    scratch_types=[
        pltpu.VMEM((CHUNK,), jnp.int32),
        pltpu.VMEM((CHUNK,), jnp.int32),
        pltpu.VMEM((CHUNK,), jnp.int32),
        pltpu.VMEM((CHUNK,), jnp.int32),
        pltpu.VMEM((CHUNK,), jnp.int32),
        pltpu.VMEM((CHUNK,), jnp.int32),
        pltpu.VMEM((CHUNK,), jnp.int32),
        pltpu.VMEM((CHUNK,), jnp.int32),
        pltpu.VMEM((CHUNK, D), jnp.float32),
        pltpu.VMEM((CHUNK, D), jnp.float32),
        pltpu.VMEM_SHARED((HALF, D), jnp.float32),
        pltpu.VMEM_SHARED((A_PAD, D), jnp.float32),
        pltpu.SemaphoreType.DMA,
        pltpu.SemaphoreType.DMA,
        pltpu.SemaphoreType.DMA,
        pltpu.SemaphoreType.DMA,
        pltpu.SemaphoreType.DMA,
        pltpu.SemaphoreType.DMA,
    ],
  )


ROW_BLK = 2000


def _mlp_body(h_ref, p_ref, w0_ref, b0_ref, w1_ref, b1_ref, o_ref):
    t = h_ref[...] + p_ref[0] + p_ref[1]
    t = jnp.maximum(
        jnp.dot(t, w0_ref[...], preferred_element_type=jnp.float32) + b0_ref[...], 0.0)
    t = jnp.maximum(
        jnp.dot(t, w1_ref[...], preferred_element_type=jnp.float32) + b1_ref[...], 0.0)
    o_ref[...] = t


def _mlp(h, parts, w0, b0, w1, b1):
    n_blk = N_NODES // ROW_BLK
    return pl.pallas_call(
        _mlp_body,
        grid=(n_blk,),
        in_specs=[
            pl.BlockSpec((ROW_BLK, D), lambda i: (i, 0)),
            pl.BlockSpec((NC, ROW_BLK, D), lambda i: (0, i, 0)),
            pl.BlockSpec((D, D), lambda i: (0, 0)),
            pl.BlockSpec((1, D), lambda i: (0, 0)),
            pl.BlockSpec((D, D), lambda i: (0, 0)),
            pl.BlockSpec((1, D), lambda i: (0, 0)),
        ],
        out_specs=pl.BlockSpec((ROW_BLK, D), lambda i: (i, 0)),
        out_shape=jax.ShapeDtypeStruct((N_NODES, D), jnp.float32),
    )(h, parts, w0, b0, w1, b1)


def _pool_body(h_ref, batch_ref, wl_ref, bl_ref, o_ref, acc_ref):
    i = pl.program_id(0)

    @pl.when(i == 0)
    def _init():
        acc_ref[...] = jnp.full((N_GRAPHS, D), -jnp.inf, jnp.float32)

    bcol = batch_ref[0]
    h = h_ref[...]
    rows = []
    for g in range(N_GRAPHS):
        m = bcol == g
        rows.append(jnp.max(jnp.where(m, h, -jnp.inf), axis=0)[None, :])
    acc_ref[...] = jnp.maximum(acc_ref[...], jnp.concatenate(rows, axis=0))

    @pl.when(i == pl.num_programs(0) - 1)
    def _fin():
        pooled = acc_ref[...]
        logits = jnp.dot(pooled, wl_ref[...],
                         preferred_element_type=jnp.float32) + bl_ref[...]
        mx = jnp.max(logits, axis=-1, keepdims=True)
        lse = jnp.log(jnp.sum(jnp.exp(logits - mx), axis=-1, keepdims=True)) + mx
        o_ref[...] = logits - lse


def _pool(h, batch3, wl, bl):
    n_blk = N_NODES // ROW_BLK
    return pl.pallas_call(
        _pool_body,
        grid=(n_blk,),
        in_specs=[
            pl.BlockSpec((ROW_BLK, D), lambda i: (i, 0)),
            pl.BlockSpec((1, ROW_BLK, 1), lambda i: (i, 0, 0)),
            pl.BlockSpec((D, N_CLASSES), lambda i: (0, 0)),
            pl.BlockSpec((1, N_CLASSES), lambda i: (0, 0)),
        ],
        out_specs=pl.BlockSpec((N_GRAPHS, N_CLASSES), lambda i: (0, 0)),
        out_shape=jax.ShapeDtypeStruct((N_GRAPHS, N_CLASSES), jnp.float32),
        scratch_shapes=[pltpu.VMEM((N_GRAPHS, D), jnp.float32)],
    )(h, batch3, wl, bl)


def kernel(x, edge_index, batch, W, b, gamma, beta, Wl, bl):
    s = gamma * (1.0 / jnp.sqrt(1.0 + BN_EPS))
    Wf = W * s[:, None, :]
    bf = (b * s + beta).reshape(6, 1, D)

    pad = E_PAD - N_EDGES
    srcp = jnp.concatenate(
        [edge_index[0], jnp.zeros((pad,), jnp.int32)]).reshape(E_PAD // CHUNK, CHUNK)
    dstp = jnp.concatenate(
        [edge_index[1], jnp.full((pad,), DUMP, jnp.int32)]).reshape(E_PAD // CHUNK, CHUNK)
    batch3 = batch.reshape(N_NODES // ROW_BLK, ROW_BLK, 1)
    hrow_pad = jnp.zeros((NC * HALF - N_NODES, D), jnp.float32)

    h = x
    for layer in range(3):
        h_pad = jnp.concatenate([h, hrow_pad])
        parts = _get_segsum()(h_pad, srcp, dstp)
        h = _mlp(h, parts, Wf[2 * layer], bf[2 * layer],
                 Wf[2 * layer + 1], bf[2 * layer + 1])
    return _pool(h, batch3, Wl, bl.reshape(1, N_CLASSES))


# speedup vs baseline: 1.2126x; 1.2126x over previous
import functools

import jax
import jax.numpy as jnp
from jax import lax
from jax.experimental import pallas as pl
from jax.experimental.pallas import tpu as pltpu
from jax.experimental.pallas import tpu_sc as plsc

N_NODES = 10000
N_EDGES = 320000
D = 128
N_GRAPHS = 64
N_CLASSES = 10
BN_EPS = 1e-5

NC = 2
NS = 16
NW = NC * NS
N_PAD = 10240
E_PER_W = 10240
E_PAD = NW * E_PER_W
CHUNK = 128
NSPLIT = 4
CHUNKS_PER_W = E_PER_W // CHUNK
ROWS_PER_S = N_PAD // NS


def _segsum_body(h_hbm, src_hbm, dst_hbm, out_hbm,
                 srcv, dst0, dst1, rows0, rows1, acc,
                 gsem0, gsem1, ssem0, ssem1, isem0, isem1):
    c = lax.axis_index("c")
    s = lax.axis_index("s")
    w = s * NC + c
    base = w * CHUNKS_PER_W
    last = CHUNKS_PER_W - 1
    sub = CHUNK // NSPLIT

    def _gather(j, buf, sem):
        for q in range(NSPLIT):
            pltpu.async_copy(h_hbm.at[srcv.at[j, pl.ds(q * sub, sub)]],
                             buf.at[pl.ds(q * sub, sub), :], sem)

    def _gwait(buf, sem):
        pltpu.make_async_copy(h_hbm.at[pl.ds(0, CHUNK), :], buf, sem).wait()

    zv = jnp.zeros((16,), jnp.float32)

    def _zrow(r, carry):
        for k in range(D // 16):
            rows0[r, pl.ds(k * 16, 16)] = zv
        return carry

    lax.fori_loop(0, CHUNK, _zrow, 0)

    def _zacc(j, carry):
        pltpu.sync_copy(rows0, acc.at[pl.ds(s * ROWS_PER_S + j * CHUNK, CHUNK), :])
        return carry

    lax.fori_loop(0, ROWS_PER_S // CHUNK, _zacc, 0)
    plsc.subcore_barrier()

    dsts = (dst0, dst1)
    rows = (rows0, rows1)
    gsems = (gsem0, gsem1)
    ssems = (ssem0, ssem1)
    isems = (isem0, isem1)

    pltpu.sync_copy(src_hbm.at[pl.ds(base, CHUNKS_PER_W), :], srcv)

    pltpu.async_copy(dst_hbm.at[base], dst0, isem0)
    _gather(0, rows0, gsem0)
    pltpu.async_copy(dst_hbm.at[base + 1], dst1, isem1)
    _gather(1, rows1, gsem1)
    _gwait(rows0, gsem0)
    pltpu.make_async_copy(dst_hbm.at[base], dst0, isem0).wait()
    pltpu.async_copy(rows0, acc.at[dst0], ssem0, add=True)

    def _pair(j2, carry):
        for u in range(2):
            j = 2 * j2 + 1 + u
            b = (1 + u) % 2
            nb = 1 - b
            pltpu.make_async_copy(h_hbm.at[pl.ds(0, CHUNK), :],
                                  rows[nb], ssems[nb]).wait()
            pltpu.async_copy(dst_hbm.at[base + j + 1], dsts[nb], isems[nb])
            _gather(j + 1, rows[nb], gsems[nb])
            _gwait(rows[b], gsems[b])
            pltpu.make_async_copy(dst_hbm.at[base + j], dsts[b], isems[b]).wait()
            pltpu.async_copy(rows[b], acc.at[dsts[b]], ssems[b], add=True)
        return carry

    lax.fori_loop(0, (CHUNKS_PER_W - 2) // 2, _pair, 0)
    pltpu.make_async_copy(h_hbm.at[pl.ds(0, CHUNK), :], rows0, ssem0).wait()
    _gwait(rows1, gsem1)
    pltpu.make_async_copy(dst_hbm.at[base + last], dst1, isem1).wait()
    pltpu.async_copy(rows1, acc.at[dst1], ssem1, add=True)
    pltpu.make_async_copy(h_hbm.at[pl.ds(0, CHUNK), :], rows1, ssem1).wait()
    plsc.subcore_barrier()

    def _out(j, carry):
        r0 = s * ROWS_PER_S + j * CHUNK
        pltpu.sync_copy(acc.at[pl.ds(r0, CHUNK), :], rows0)
        pltpu.sync_copy(rows0, out_hbm.at[c, pl.ds(r0, CHUNK), :])
        return carry

    lax.fori_loop(0, ROWS_PER_S // CHUNK, _out, 0)


@functools.lru_cache(maxsize=1)
def _get_segsum():
  return pl.kernel(
    _segsum_body,
    mesh=plsc.VectorSubcoreMesh(core_axis_name="c", subcore_axis_name="s"),
    out_type=jax.ShapeDtypeStruct((NC, N_PAD, D), jnp.float32),
    scratch_types=[
        pltpu.VMEM((CHUNKS_PER_W, CHUNK), jnp.int32),
        pltpu.VMEM((CHUNK,), jnp.int32),
        pltpu.VMEM((CHUNK,), jnp.int32),
        pltpu.VMEM((CHUNK, D), jnp.float32),
        pltpu.VMEM((CHUNK, D), jnp.float32),
        pltpu.VMEM_SHARED((N_PAD, D), jnp.float32),
        pltpu.SemaphoreType.DMA,
        pltpu.SemaphoreType.DMA,
        pltpu.SemaphoreType.DMA,
        pltpu.SemaphoreType.DMA,
        pltpu.SemaphoreType.DMA,
        pltpu.SemaphoreType.DMA,
    ],
  )


ROW_BLK = 2000


def _mlp_body(h_ref, p_ref, w0_ref, b0_ref, w1_ref, b1_ref, o_ref):
    t = h_ref[...] + p_ref[0] + p_ref[1]
    t = jnp.maximum(
        jnp.dot(t, w0_ref[...], preferred_element_type=jnp.float32) + b0_ref[...], 0.0)
    t = jnp.maximum(
        jnp.dot(t, w1_ref[...], preferred_element_type=jnp.float32) + b1_ref[...], 0.0)
    o_ref[...] = t


def _mlp(h, parts, w0, b0, w1, b1):
    n_blk = N_NODES // ROW_BLK
    return pl.pallas_call(
        _mlp_body,
        grid=(n_blk,),
        in_specs=[
            pl.BlockSpec((ROW_BLK, D), lambda i: (i, 0)),
            pl.BlockSpec((NC, ROW_BLK, D), lambda i: (0, i, 0)),
            pl.BlockSpec((D, D), lambda i: (0, 0)),
            pl.BlockSpec((1, D), lambda i: (0, 0)),
            pl.BlockSpec((D, D), lambda i: (0, 0)),
            pl.BlockSpec((1, D), lambda i: (0, 0)),
        ],
        out_specs=pl.BlockSpec((ROW_BLK, D), lambda i: (i, 0)),
        out_shape=jax.ShapeDtypeStruct((N_NODES, D), jnp.float32),
    )(h, parts, w0, b0, w1, b1)


def _pool_body(h_ref, batch_ref, wl_ref, bl_ref, o_ref, acc_ref):
    i = pl.program_id(0)

    @pl.when(i == 0)
    def _init():
        acc_ref[...] = jnp.full((N_GRAPHS, D), -jnp.inf, jnp.float32)

    bcol = batch_ref[0]
    h = h_ref[...]
    rows = []
    for g in range(N_GRAPHS):
        m = bcol == g
        rows.append(jnp.max(jnp.where(m, h, -jnp.inf), axis=0)[None, :])
    acc_ref[...] = jnp.maximum(acc_ref[...], jnp.concatenate(rows, axis=0))

    @pl.when(i == pl.num_programs(0) - 1)
    def _fin():
        pooled = acc_ref[...]
        logits = jnp.dot(pooled, wl_ref[...],
                         preferred_element_type=jnp.float32) + bl_ref[...]
        mx = jnp.max(logits, axis=-1, keepdims=True)
        lse = jnp.log(jnp.sum(jnp.exp(logits - mx), axis=-1, keepdims=True)) + mx
        o_ref[...] = logits - lse


def _pool(h, batch3, wl, bl):
    n_blk = N_NODES // ROW_BLK
    return pl.pallas_call(
        _pool_body,
        grid=(n_blk,),
        in_specs=[
            pl.BlockSpec((ROW_BLK, D), lambda i: (i, 0)),
            pl.BlockSpec((1, ROW_BLK, 1), lambda i: (i, 0, 0)),
            pl.BlockSpec((D, N_CLASSES), lambda i: (0, 0)),
            pl.BlockSpec((1, N_CLASSES), lambda i: (0, 0)),
        ],
        out_specs=pl.BlockSpec((N_GRAPHS, N_CLASSES), lambda i: (0, 0)),
        out_shape=jax.ShapeDtypeStruct((N_GRAPHS, N_CLASSES), jnp.float32),
        scratch_shapes=[pltpu.VMEM((N_GRAPHS, D), jnp.float32)],
    )(h, batch3, wl, bl)


def kernel(x, edge_index, batch, W, b, gamma, beta, Wl, bl):
    s = gamma * (1.0 / jnp.sqrt(1.0 + BN_EPS))
    Wf = W * s[:, None, :]
    bf = (b * s + beta).reshape(6, 1, D)

    pad = E_PAD - N_EDGES
    srcp = jnp.concatenate(
        [edge_index[0], jnp.zeros((pad,), jnp.int32)]).reshape(E_PAD // CHUNK, CHUNK)
    dstp = jnp.concatenate(
        [edge_index[1], jnp.full((pad,), N_NODES, jnp.int32)]).reshape(E_PAD // CHUNK, CHUNK)
    batch3 = batch.reshape(N_NODES // ROW_BLK, ROW_BLK, 1)

    h = x
    for layer in range(3):
        parts = _get_segsum()(h, srcp, dstp)
        h = _mlp(h, parts, Wf[2 * layer], bf[2 * layer],
                 Wf[2 * layer + 1], bf[2 * layer + 1])
    return _pool(h, batch3, Wl, bl.reshape(1, N_CLASSES))
